# Initial kernel scaffold; baseline (speedup 1.0000x reference)
import jax
import jax.numpy as jnp
from jax.experimental import pallas as pl


def kernel(positions, obs_pos, poly_dic, graph_dic, S_poly, S_graph, alpha_poly, alpha_graph):
    raise NotImplementedError("write your pallas kernel here")


# speedup vs baseline: 1.2655x; 1.2655x over previous
import jax
import jax.numpy as jnp
from jax.experimental import pallas as pl
from jax.experimental.pallas import tpu as pltpu

_B = 256
_N = 4096
_DD = 256 * 256
_TILE = 4096
_NT = _DD // _TILE


def _opd_kernel(pos_ref, obs_t_ref, poly_ref, graph_ref, ap_ref, ag_ref,
                sp_ref, sg_ref, out_ref, cp_ref, cg_ref):
    i = pl.program_id(0)

    @pl.when(i == 0)
    def _stage_a():
        px = pos_ref[:, 0:1]
        py = pos_ref[:, 1:2]
        ox = obs_t_ref[0:1, :]
        oy = obs_t_ref[1:2, :]
        d = (px - ox) ** 2 + (py - oy) ** 2
        md = jnp.min(d, axis=1, keepdims=True)
        iota = jax.lax.broadcasted_iota(jnp.int32, (_B, _N), 1)
        idx = jnp.min(jnp.where(d == md, iota, _N), axis=1, keepdims=True)
        onehot = (iota == idx).astype(jnp.float32)
        gp = jnp.dot(onehot, poly_ref[...], preferred_element_type=jnp.float32)
        gg = jnp.dot(onehot, graph_ref[...], preferred_element_type=jnp.float32)
        cp_ref[...] = jnp.dot(gp, ap_ref[...], preferred_element_type=jnp.float32)
        cg_ref[...] = jnp.dot(gg, ag_ref[...], preferred_element_type=jnp.float32)

    out_ref[...] = (
        jnp.dot(cp_ref[...], sp_ref[...], preferred_element_type=jnp.float32)
        + jnp.dot(cg_ref[...], sg_ref[...], preferred_element_type=jnp.float32)
    )


def kernel(positions, obs_pos, poly_dic, graph_dic, S_poly, S_graph,
           alpha_poly, alpha_graph):
    pf = alpha_poly.shape[1]
    gf = alpha_graph.shape[1]
    sp2 = S_poly.reshape(pf, _DD)
    sg2 = S_graph.reshape(gf, _DD)
    obs_t = obs_pos.T

    out = pl.pallas_call(
        _opd_kernel,
        grid=(_NT,),
        in_specs=[
            pl.BlockSpec((_B, 2), lambda i: (0, 0)),
            pl.BlockSpec((2, _N), lambda i: (0, 0)),
            pl.BlockSpec(poly_dic.shape, lambda i: (0, 0)),
            pl.BlockSpec(graph_dic.shape, lambda i: (0, 0)),
            pl.BlockSpec(alpha_poly.shape, lambda i: (0, 0)),
            pl.BlockSpec(alpha_graph.shape, lambda i: (0, 0)),
            pl.BlockSpec((pf, _TILE), lambda i: (0, i)),
            pl.BlockSpec((gf, _TILE), lambda i: (0, i)),
        ],
        out_specs=pl.BlockSpec((_B, _TILE), lambda i: (0, i)),
        out_shape=jax.ShapeDtypeStruct((_B, _DD), jnp.float32),
        scratch_shapes=[
            pltpu.VMEM((_B, pf), jnp.float32),
            pltpu.VMEM((_B, gf), jnp.float32),
        ],
    )(positions, obs_t, poly_dic, graph_dic, alpha_poly, alpha_graph, sp2, sg2)

    opd_maps = out.reshape(_B, 256, 256)
    return (opd_maps, alpha_graph)
